# Initial kernel scaffold; baseline (speedup 1.0000x reference)
import jax
import jax.numpy as jnp
from jax.experimental import pallas as pl


def kernel(distance, lane, wheel_feat, sensor_feat, norm_target, damper_idx, params):
    raise NotImplementedError("write your pallas kernel here")


# speedup vs baseline: 1.2112x; 1.2112x over previous
import numpy as np
import jax
import jax.numpy as jnp
from jax import lax
from jax.experimental import pallas as pl
from jax.experimental.pallas import tpu as pltpu

_WS = 2048
_HEADS = 4
_DH = 7
_SCALE = 28.0 ** -0.5
_CONN_NORM = np.float32(1.0) / np.float32(np.sqrt(np.float32(2.0))) * np.float32(0.5)

_EDGE_NAMES = ["front", "rear", "right", "left"]
_EDGE_LIST = [([2, 3], [0, 1]), ([0, 1], [2, 3]), ([0, 2], [1, 3]), ([1, 3], [0, 2])]


def _adj(src, dst):
    a = np.zeros((4, 4), np.float32)
    for s, d in zip(src, dst):
        a[d, s] = 1.0
    return a


_A_W = np.stack([_adj(s, d) for s, d in _EDGE_LIST])


def _prep_weights(p, damper_idx):
    f32 = jnp.float32
    bf16 = jnp.bfloat16
    eye4 = jnp.eye(4, dtype=f32)

    bd1 = jnp.kron(eye4, p["few_W1"]).astype(bf16)
    bt1 = jnp.tile(p["few_b1"], 4)[None, :]
    bd2 = jnp.kron(eye4, p["few_W2"]).astype(bf16)
    bt2 = jnp.tile(p["few_b2"], 4)[None, :]

    c1 = jnp.stack([p[f"c1_{e}_W"] for e in _EDGE_NAMES])
    m1 = jnp.einsum("eij,ekl->jkil", jnp.asarray(_A_W), c1).reshape(60, 32)
    b1 = jnp.tile(sum(p[f"c1_{e}_b"] for e in _EDGE_NAMES), 4)[None, :]

    c2 = jnp.stack([p[f"c2_{e}_W"] for e in _EDGE_NAMES])
    m2 = jnp.einsum("eij,ekl->jkil", jnp.asarray(_A_W), c2).reshape(32, 16)
    b2 = jnp.tile(sum(p[f"c2_{e}_b"] for e in _EDGE_NAMES), 4)[None, :]

    front = dict(
        bd1=bd1, bt1=bt1, bd2=bd2, bt2=bt2,
        m1=m1.astype(bf16), b1=b1, m2=m2.astype(bf16), b2=b2,
        c2c=p["c2_connect_W"].astype(bf16), b2c=p["c2_connect_b"][None, :],
        ntw1=p["nt_W1"].astype(bf16), ntb1=p["nt_b1"][None, :],
        ntw2=p["nt_W2"].astype(bf16), ntb2=p["nt_b2"][None, :],
        rse=p["rse_W"].astype(bf16), rseb=p["rse_b"][None, :],
        dmp=p["damper_E"][damper_idx][None, :],
    )

    def stk(f, dt=bf16):
        return jnp.stack([f(L) for L in p["layers"]]).astype(dt)

    tw = dict(
        ln1g=stk(lambda L: L["ln1_g"][None, :], f32),
        ln1b=stk(lambda L: L["ln1_b"][None, :], f32),
        wq=stk(lambda L: L["qkv_W"][:, 0:28]),
        wk=stk(lambda L: L["qkv_W"][:, 28:56]),
        wv=stk(lambda L: L["qkv_W"][:, 56:84]),
        ow=stk(lambda L: L["out_W"]),
        ob=stk(lambda L: L["out_b"][None, :], f32),
        ln2g=stk(lambda L: L["ln2_g"][None, :], f32),
        ln2b=stk(lambda L: L["ln2_b"][None, :], f32),
        f1=stk(lambda L: L["ff_W1"]),
        fb1=stk(lambda L: L["ff_b1"][None, :], f32),
        f2=stk(lambda L: L["ff_W2"]),
        fb2=stk(lambda L: L["ff_b2"][None, :], f32),
    )

    head = dict(
        w1p=p["head_W1"].reshape(_WS, 28, 128).transpose(1, 0, 2).astype(bf16),
        b1=p["head_b1"][None, :],
        w2=p["head_W2"].astype(bf16),
        b2=p["head_b2"][None, :],
        w3=p["head_W3"].astype(bf16),
        b3=p["head_b3"][None, :],
    )
    return front, tw, head


def _bdot(a, b):
    return jnp.dot(a.astype(jnp.bfloat16), b, preferred_element_type=jnp.float32)


def _layer_norm(x, g, b):
    mu = jnp.mean(x, axis=-1, keepdims=True)
    var = jnp.mean((x - mu) ** 2, axis=-1, keepdims=True)
    return (x - mu) / jnp.sqrt(var + 1e-5) * g + b


def _main_body(dist, wf, nt,
               bd1, bt1, bd2, bt2, m1, b1, m2, b2, c2c, b2c,
               ntw1, ntb1, ntw2, ntb2, rse, rseb, dmp,
               ln1g, ln1b, wq, wk, wv, ow, ob, ln2g, ln2b, f1, fb1, f2, fb2,
               x_out):
    f32 = jnp.float32
    bf16 = jnp.bfloat16
    lr = lambda t: jnp.where(t >= 0, t, 0.01 * t)

    u = lr(_bdot(wf[...], bd1[...]) + bt1[...])
    w = _bdot(u, bd2[...]) + bt2[...]
    h1 = lr(_bdot(w, m1[...]) + b1[...])
    h2w = _bdot(h1, m2[...]) + b2[...]
    aggc = (h1[:, 0:8] + h1[:, 8:16] + h1[:, 16:24] + h1[:, 24:32]) * _CONN_NORM
    h2s = _bdot(aggc, c2c[...]) + b2c[...]
    ntv = jnp.maximum(_bdot(nt[...], ntw1[...]) + ntb1[...], 0.0)
    nt5 = _bdot(ntv, ntw2[...]) + ntb2[...]
    x37 = jnp.concatenate(
        [dist[...], h2w, h2s, h2s, nt5,
         jnp.broadcast_to(dmp[...], (_WS, 7))], axis=1)
    x = _bdot(x37, rse[...]) + rseb[...]

    for L in range(3):
        y = _layer_norm(x, ln1g[L], ln1b[L])
        yb = y.astype(bf16)
        q = jnp.dot(yb, wq[L], preferred_element_type=f32)
        k = jnp.dot(yb, wk[L], preferred_element_type=f32)
        v = jnp.dot(yb, wv[L], preferred_element_type=f32).astype(bf16)
        outs = []
        for h in range(_HEADS):
            sl = slice(h * _DH, (h + 1) * _DH)
            dots = lax.dot_general(q[:, sl].astype(bf16), k[:, sl].astype(bf16),
                                   (((1,), (1,)), ((), ())),
                                   preferred_element_type=f32) * _SCALE
            m = jnp.max(dots, axis=-1, keepdims=True)
            e = jnp.exp(dots - m)
            s = jnp.sum(e, axis=-1, keepdims=True)
            a = (e / s).astype(bf16)
            outs.append(jnp.dot(a, v[:, sl], preferred_element_type=f32))
        o = jnp.concatenate(outs, axis=1)
        x = x + _bdot(o, ow[L]) + ob[L]
        y2 = _layer_norm(x, ln2g[L], ln2b[L])
        g = _bdot(y2, f1[L]) + fb1[L]
        g = 0.5 * g * (1.0 + lax.erf(g / np.float32(np.sqrt(2.0))))
        x = x + _bdot(g, f2[L]) + fb2[L]

    x_out[...] = x


_HEAD_CHUNK = 256


def _head_body(x_ref, w1_ref, hb1, hw2, hb2, hw3, hb3, out_ref, acc_ref):
    i = pl.program_id(0)
    f32 = jnp.float32

    @pl.when(i == 0)
    def _():
        acc_ref[...] = jnp.zeros_like(acc_ref)

    xb = x_ref[...].astype(jnp.bfloat16).astype(f32)
    total = jnp.zeros((1, 128), f32)
    for c in range(28):
        total = total + jnp.sum(xb[:, c:c + 1] * w1_ref[c].astype(f32),
                                axis=0, keepdims=True)
    acc_ref[...] += total

    @pl.when(i == pl.num_programs(0) - 1)
    def _():
        r = jnp.maximum(acc_ref[...] + hb1[...], 0.0)
        r = jnp.maximum(_bdot(r, hw2[...]) + hb2[...], 0.0)
        out_ref[...] = _bdot(r, hw3[...]) + hb3[...]


def kernel(distance, lane, wheel_feat, sensor_feat, norm_target, damper_idx,
           params):
    del lane, sensor_feat
    front, tw, head = _prep_weights(params, damper_idx)

    wf160 = wheel_feat.reshape(_WS, 160)
    nt20 = norm_target.reshape(_WS, 20)

    x = pl.pallas_call(
        _main_body,
        out_shape=jax.ShapeDtypeStruct((_WS, 28), jnp.float32),
    )(distance, wf160, nt20,
      front["bd1"], front["bt1"], front["bd2"], front["bt2"],
      front["m1"], front["b1"], front["m2"], front["b2"],
      front["c2c"], front["b2c"],
      front["ntw1"], front["ntb1"], front["ntw2"], front["ntb2"],
      front["rse"], front["rseb"], front["dmp"],
      tw["ln1g"], tw["ln1b"], tw["wq"], tw["wk"], tw["wv"], tw["ow"], tw["ob"],
      tw["ln2g"], tw["ln2b"], tw["f1"], tw["fb1"], tw["f2"], tw["fb2"])

    n_chunks = _WS // _HEAD_CHUNK
    out = pl.pallas_call(
        _head_body,
        grid=(n_chunks,),
        in_specs=[
            pl.BlockSpec((_HEAD_CHUNK, 28), lambda i: (i, 0)),
            pl.BlockSpec((28, _HEAD_CHUNK, 128), lambda i: (0, i, 0)),
            pl.BlockSpec((1, 128), lambda i: (0, 0)),
            pl.BlockSpec((128, 32), lambda i: (0, 0)),
            pl.BlockSpec((1, 32), lambda i: (0, 0)),
            pl.BlockSpec((32, 4), lambda i: (0, 0)),
            pl.BlockSpec((1, 4), lambda i: (0, 0)),
        ],
        out_specs=pl.BlockSpec((1, 4), lambda i: (0, 0)),
        out_shape=jax.ShapeDtypeStruct((1, 4), jnp.float32),
        scratch_shapes=[pltpu.VMEM((1, 128), jnp.float32)],
    )(x, head["w1p"], head["b1"], head["w2"], head["b2"], head["w3"],
      head["b3"])
    return out


# speedup vs baseline: 1.3193x; 1.0892x over previous
import numpy as np
import jax
import jax.numpy as jnp
from jax import lax
from jax.experimental import pallas as pl
from jax.experimental.pallas import tpu as pltpu

_WS = 2048
_HEADS = 4
_DH = 7
_SCALE = 28.0 ** -0.5
_CONN_NORM = np.float32(1.0) / np.float32(np.sqrt(np.float32(2.0))) * np.float32(0.5)

_EDGE_NAMES = ["front", "rear", "right", "left"]
_EDGE_LIST = [([2, 3], [0, 1]), ([0, 1], [2, 3]), ([0, 2], [1, 3]), ([1, 3], [0, 2])]


def _adj(src, dst):
    a = np.zeros((4, 4), np.float32)
    for s, d in zip(src, dst):
        a[d, s] = 1.0
    return a


_A_W = np.stack([_adj(s, d) for s, d in _EDGE_LIST])


def _bf(t):
    return t.astype(jnp.bfloat16)


def _dot(a, b):
    return jnp.dot(_bf(a), _bf(b), preferred_element_type=jnp.float32)


def _layer_norm(x, g, b):
    mu = jnp.mean(x, axis=-1, keepdims=True)
    var = jnp.mean((x - mu) ** 2, axis=-1, keepdims=True)
    return (x - mu) / jnp.sqrt(var + 1e-5) * g + b


def _tile4(b):
    r = b.reshape(1, -1)
    return jnp.concatenate([r, r, r, r], axis=1)


def _main_body(dist, wf, nt,
               bd1, bd2, m1, m2, few_b1, few_b2, c1bsum, c2bsum, c2c, c2cb,
               ntw1, ntb1, ntw2, ntb2, rse, rseb, dmp,
               l0, l1, l2,
               x_out):
    f32 = jnp.float32
    bf16 = jnp.bfloat16
    lr = lambda t: jnp.where(t >= 0, t, 0.01 * t)

    u = lr(_dot(wf[...], bd1[...]) + _tile4(few_b1[...]))
    w = _dot(u, bd2[...]) + _tile4(few_b2[...])
    h1 = lr(_dot(w, m1[...]) + _tile4(c1bsum[...]))
    h2w = _dot(h1, m2[...]) + _tile4(c2bsum[...])
    aggc = (h1[:, 0:8] + h1[:, 8:16] + h1[:, 16:24] + h1[:, 24:32]) * _CONN_NORM
    h2s = _dot(aggc, c2c[...]) + c2cb[...].reshape(1, -1)
    ntv = jnp.maximum(_dot(nt[...], ntw1[...]) + ntb1[...].reshape(1, -1), 0.0)
    nt5 = _dot(ntv, ntw2[...]) + ntb2[...].reshape(1, -1)
    x37 = jnp.concatenate(
        [dist[...], h2w, h2s, h2s, nt5,
         jnp.broadcast_to(dmp[...], (_WS, 7))], axis=1)
    x = _dot(x37, rse[...]) + rseb[...].reshape(1, -1)

    for (ln1g, ln1b, qkvw, ow, ob, ln2g, ln2b, f1, fb1, f2, fb2) in (l0, l1, l2):
        y = _layer_norm(x, ln1g[...].reshape(1, -1), ln1b[...].reshape(1, -1))
        qkv = _dot(y, qkvw[...])
        outs = []
        for h in range(_HEADS):
            q = _bf(qkv[:, h * _DH:(h + 1) * _DH])
            k = _bf(qkv[:, 28 + h * _DH:28 + (h + 1) * _DH])
            v = _bf(qkv[:, 56 + h * _DH:56 + (h + 1) * _DH])
            dots = lax.dot_general(q, k, (((1,), (1,)), ((), ())),
                                   preferred_element_type=f32) * _SCALE
            m = jnp.max(dots, axis=-1, keepdims=True)
            e = jnp.exp(dots - m)
            s = jnp.sum(e, axis=-1, keepdims=True)
            a = (e / s).astype(bf16)
            outs.append(jnp.dot(a, v, preferred_element_type=f32))
        o = jnp.concatenate(outs, axis=1)
        x = x + _dot(o, ow[...]) + ob[...].reshape(1, -1)
        y2 = _layer_norm(x, ln2g[...].reshape(1, -1), ln2b[...].reshape(1, -1))
        g = _dot(y2, f1[...]) + fb1[...].reshape(1, -1)
        g = 0.5 * g * (1.0 + lax.erf(g / np.float32(np.sqrt(2.0))))
        x = x + _dot(g, f2[...]) + fb2[...].reshape(1, -1)

    x_out[...] = x


_HEAD_CHUNK = 256


def _head_body(x_ref, w1_ref, hb1, hw2, hb2, hw3, hb3, out_ref, g_ref):
    i = pl.program_id(0)
    f32 = jnp.float32

    @pl.when(i == 0)
    def _():
        g_ref[...] = jnp.zeros_like(g_ref)

    xt = _bf(x_ref[...]).T
    g_ref[...] += jnp.dot(xt, w1_ref[...], preferred_element_type=f32)

    @pl.when(i == pl.num_programs(0) - 1)
    def _():
        g = g_ref[...]
        acc = jnp.zeros((1, 128), f32)
        for c in range(28):
            acc = acc + g[c:c + 1, c * 128:(c + 1) * 128]
        r = jnp.maximum(acc + hb1[...].reshape(1, -1), 0.0)
        r = jnp.maximum(_dot(r, hw2[...]) + hb2[...].reshape(1, -1), 0.0)
        out_ref[...] = _dot(r, hw3[...]) + hb3[...].reshape(1, -1)


def kernel(distance, lane, wheel_feat, sensor_feat, norm_target, damper_idx,
           params):
    del lane, sensor_feat
    p = params
    f32 = jnp.float32
    eye4 = jnp.eye(4, dtype=f32)

    bd1 = jnp.kron(eye4, p["few_W1"])
    bd2 = jnp.kron(eye4, p["few_W2"])
    c1 = jnp.stack([p[f"c1_{e}_W"] for e in _EDGE_NAMES])
    m1 = jnp.einsum("eij,ekl->jkil", jnp.asarray(_A_W), c1).reshape(60, 32)
    c2 = jnp.stack([p[f"c2_{e}_W"] for e in _EDGE_NAMES])
    m2 = jnp.einsum("eij,ekl->jkil", jnp.asarray(_A_W), c2).reshape(32, 16)
    c1bsum = sum(p[f"c1_{e}_b"] for e in _EDGE_NAMES)
    c2bsum = sum(p[f"c2_{e}_b"] for e in _EDGE_NAMES)
    dmp = p["damper_E"][damper_idx][None, :]
    w1r = p["head_W1"].reshape(_WS, 28 * 128).astype(jnp.bfloat16)

    wf160 = wheel_feat.reshape(_WS, 160)
    nt20 = norm_target.reshape(_WS, 20)

    layer_args = []
    for L in p["layers"]:
        layer_args += [L["ln1_g"], L["ln1_b"], L["qkv_W"], L["out_W"],
                       L["out_b"], L["ln2_g"], L["ln2_b"], L["ff_W1"],
                       L["ff_b1"], L["ff_W2"], L["ff_b2"]]

    def body(*refs):
        n = 20
        per = 11
        _main_body(*refs[:n],
                   refs[n:n + per], refs[n + per:n + 2 * per],
                   refs[n + 2 * per:n + 3 * per], refs[-1])

    x = pl.pallas_call(
        body,
        out_shape=jax.ShapeDtypeStruct((_WS, 28), f32),
    )(distance, wf160, nt20,
      bd1, bd2, m1, m2, p["few_b1"], p["few_b2"], c1bsum, c2bsum,
      p["c2_connect_W"], p["c2_connect_b"],
      p["nt_W1"], p["nt_b1"], p["nt_W2"], p["nt_b2"],
      p["rse_W"], p["rse_b"], dmp,
      *layer_args)

    n_chunks = _WS // _HEAD_CHUNK
    out = pl.pallas_call(
        _head_body,
        grid=(n_chunks,),
        in_specs=[
            pl.BlockSpec((_HEAD_CHUNK, 28), lambda i: (i, 0)),
            pl.BlockSpec((_HEAD_CHUNK, 28 * 128), lambda i: (i, 0)),
            pl.BlockSpec((128,), lambda i: (0,)),
            pl.BlockSpec((128, 32), lambda i: (0, 0)),
            pl.BlockSpec((32,), lambda i: (0,)),
            pl.BlockSpec((32, 4), lambda i: (0, 0)),
            pl.BlockSpec((4,), lambda i: (0,)),
        ],
        out_specs=pl.BlockSpec((1, 4), lambda i: (0, 0)),
        out_shape=jax.ShapeDtypeStruct((1, 4), f32),
        scratch_shapes=[pltpu.VMEM((28, 28 * 128), f32)],
    )(x, w1r, p["head_b1"], p["head_W2"], p["head_b2"], p["head_W3"],
      p["head_b3"])
    return out
